# baseline (speedup 1.0000x reference)
import functools

import jax
import jax.numpy as jnp
from jax import lax
from jax.experimental import pallas as pl
from jax.experimental.pallas import tpu as pltpu
from jax.experimental.pallas import tpu_sc as plsc

N_NODES = 10000
N_EDGES = 320000
DIM = 128

NC = 2
NS = 16
E_PER_TILE = N_EDGES // (NC * NS)
CHUNK = 80
N_CHUNKS = E_PER_TILE // CHUNK
R_MAIN = 624
R_TAIL = N_NODES - NS * R_MAIN


def _sc_accumulate(feature, src, dst, zrows, zcnt, ones_v_h):
    mesh = plsc.VectorSubcoreMesh(core_axis_name="c", subcore_axis_name="s")

    @functools.partial(
        pl.kernel,
        mesh=mesh,
        out_type=[
            jax.ShapeDtypeStruct((NC, N_NODES, DIM), jnp.float32),
            jax.ShapeDtypeStruct((16, N_NODES), jnp.float32),
        ],
        scratch_types=[
            pltpu.VMEM((E_PER_TILE,), jnp.int32),
            pltpu.VMEM((E_PER_TILE,), jnp.int32),
            pltpu.VMEM((CHUNK, DIM), jnp.float32),
            pltpu.VMEM((CHUNK, DIM), jnp.float32),
            pltpu.VMEM((CHUNK,), jnp.float32),
            pltpu.VMEM_SHARED((N_NODES, DIM), jnp.float32),
            pltpu.VMEM_SHARED((N_NODES,), jnp.float32),
            pltpu.SemaphoreType.DMA,
            pltpu.SemaphoreType.DMA,
            pltpu.SemaphoreType.DMA,
            pltpu.SemaphoreType.DMA,
            pltpu.SemaphoreType.DMA,
            pltpu.SemaphoreType.DMA,
        ],
    )
    def k(feat_hbm, src_hbm, dst_hbm, zr_hbm, zc_hbm, ones_hbm,
          psum_hbm, pcnt_hbm,
          src_a, dst_a, msgs0, msgs1, ones_v, acc_s, cnt_s,
          sem0, sem1, sems0, sems1, semc0, semc1):
        c = lax.axis_index("c")
        s = lax.axis_index("s")
        tid = c * NS + s

        r0 = s * R_MAIN
        pltpu.sync_copy(zr_hbm, acc_s.at[pl.ds(r0, R_MAIN)])

        @pl.when(s == 0)
        def _():
            pltpu.sync_copy(zc_hbm, cnt_s)

        @pl.when(s == NS - 1)
        def _():
            pltpu.sync_copy(zr_hbm.at[pl.ds(0, R_TAIL)],
                            acc_s.at[pl.ds(NS * R_MAIN, R_TAIL)])

        pltpu.sync_copy(ones_hbm, ones_v)
        e0 = pl.multiple_of(tid * E_PER_TILE, 8)
        pltpu.sync_copy(src_hbm.at[pl.ds(e0, E_PER_TILE)], src_a)
        pltpu.sync_copy(dst_hbm.at[pl.ds(e0, E_PER_TILE)], dst_a)
        plsc.subcore_barrier()

        def src_at(j):
            return src_a.at[pl.ds(pl.multiple_of(j * CHUNK, 8), CHUNK)]

        def dst_at(j):
            return dst_a.at[pl.ds(pl.multiple_of(j * CHUNK, 8), CHUNK)]

        def gather_wait(j, buf, sem):
            pltpu.make_async_copy(feat_hbm.at[src_at(j)], buf, sem).wait()

        def scatter_start(j, buf, semv, semc):
            pltpu.async_copy(buf, acc_s.at[dst_at(j)], semv, add=True)
            pltpu.async_copy(ones_v, cnt_s.at[dst_at(j)], semc, add=True)

        def scatter_wait(j, buf, semv, semc):
            pltpu.make_async_copy(buf, acc_s.at[dst_at(j)], semv).wait()
            pltpu.make_async_copy(ones_v, cnt_s.at[dst_at(j)], semc).wait()

        pltpu.async_copy(feat_hbm.at[src_at(0)], msgs0, sem0)
        gather_wait(0, msgs0, sem0)
        scatter_start(0, msgs0, sems0, semc0)
        pltpu.async_copy(feat_hbm.at[src_at(1)], msgs1, sem1)

        def body(j2, carry):
            j = j2 * 2 + 1
            gather_wait(j, msgs1, sem1)
            scatter_start(j, msgs1, sems1, semc1)
            scatter_wait(j - 1, msgs0, sems0, semc0)
            pltpu.async_copy(feat_hbm.at[src_at(j + 1)], msgs0, sem0)
            gather_wait(j + 1, msgs0, sem0)
            scatter_start(j + 1, msgs0, sems0, semc0)
            scatter_wait(j, msgs1, sems1, semc1)

            @pl.when(j2 < N_CHUNKS // 2 - 1)
            def _():
                pltpu.async_copy(feat_hbm.at[src_at(j + 2)], msgs1, sem1)

            return carry

        lax.fori_loop(0, N_CHUNKS // 2, body, 0)
        scatter_wait(N_CHUNKS - 1, msgs0, sems0, semc0)

        plsc.subcore_barrier()

        pltpu.sync_copy(acc_s.at[pl.ds(r0, R_MAIN)],
                        psum_hbm.at[c, pl.ds(r0, R_MAIN)])

        @pl.when(s == 0)
        def _():
            pltpu.sync_copy(cnt_s, pcnt_hbm.at[pl.multiple_of(8 * c, 8)])

        @pl.when(s == NS - 1)
        def _():
            pltpu.sync_copy(acc_s.at[pl.ds(NS * R_MAIN, R_TAIL)],
                            psum_hbm.at[c, pl.ds(NS * R_MAIN, R_TAIL)])

    return k(feature, src, dst, zrows, zcnt, ones_v_h)


def _tc_body(p_ref, c_ref, w_ref, b_ref, o_ref):
    p = p_ref[0] + p_ref[1]
    cnt = jnp.maximum(c_ref[0] + c_ref[8], 1.0).reshape(N_NODES, 1)
    h = p / cnt
    o_ref[...] = (
        jnp.dot(h, w_ref[...], preferred_element_type=jnp.float32) + b_ref[...]
    )


def _tc_apply(psum, pcnt, Wt, b2):
    return pl.pallas_call(
        _tc_body,
        out_shape=jax.ShapeDtypeStruct((N_NODES, DIM), jnp.float32),
    )(psum, pcnt, Wt, b2)


def kernel(feature, edge_index, W, b):
    src = edge_index[0].astype(jnp.int32)
    dst = edge_index[1].astype(jnp.int32)
    zrows = jnp.zeros((R_MAIN, DIM), jnp.float32)
    zcnt = jnp.zeros((N_NODES,), jnp.float32)
    ones_v_h = jnp.ones((CHUNK,), jnp.float32)
    psum, pcnt = _sc_accumulate(feature, src, dst, zrows, zcnt, ones_v_h)
    out = _tc_apply(psum, pcnt, W.T, b.reshape(1, DIM))
    return out


# speedup vs baseline: 1.1173x; 1.1173x over previous
import functools

import jax
import jax.numpy as jnp
from jax import lax
from jax.experimental import pallas as pl
from jax.experimental.pallas import tpu as pltpu
from jax.experimental.pallas import tpu_sc as plsc

N_NODES = 10000
N_EDGES = 320000
DIM = 128

NC = 2
NS = 16
E_PER_TILE = N_EDGES // (NC * NS)
CHUNK = 112
N_CHUNKS = E_PER_TILE // CHUNK
E_TAIL = E_PER_TILE - N_CHUNKS * CHUNK
R_MAIN = 624
R_TAIL = N_NODES - NS * R_MAIN


def _sc_accumulate(feature, src, dst, zrows, zcnt, ones_v_h):
    mesh = plsc.VectorSubcoreMesh(core_axis_name="c", subcore_axis_name="s")

    @functools.partial(
        pl.kernel,
        mesh=mesh,
        out_type=[
            jax.ShapeDtypeStruct((NC, N_NODES, DIM), jnp.float32),
            jax.ShapeDtypeStruct((16, N_NODES), jnp.float32),
        ],
        scratch_types=[
            pltpu.VMEM((E_PER_TILE,), jnp.int32),
            pltpu.VMEM((E_PER_TILE,), jnp.int32),
            pltpu.VMEM((CHUNK, DIM), jnp.float32),
            pltpu.VMEM((CHUNK, DIM), jnp.float32),
            pltpu.VMEM((CHUNK,), jnp.float32),
            pltpu.VMEM_SHARED((N_NODES, DIM), jnp.float32),
            pltpu.VMEM_SHARED((N_NODES,), jnp.float32),
            pltpu.SemaphoreType.DMA,
            pltpu.SemaphoreType.DMA,
            pltpu.SemaphoreType.DMA,
            pltpu.SemaphoreType.DMA,
            pltpu.SemaphoreType.DMA,
            pltpu.SemaphoreType.DMA,
        ],
    )
    def k(feat_hbm, src_hbm, dst_hbm, zr_hbm, zc_hbm, ones_hbm,
          psum_hbm, pcnt_hbm,
          src_a, dst_a, msgs0, msgs1, ones_v, acc_s, cnt_s,
          sem0, sem1, sems0, sems1, semc0, semc1):
        c = lax.axis_index("c")
        s = lax.axis_index("s")
        tid = c * NS + s

        r0 = s * R_MAIN
        pltpu.sync_copy(zr_hbm, acc_s.at[pl.ds(r0, R_MAIN)])

        @pl.when(s == 0)
        def _():
            pltpu.sync_copy(zc_hbm, cnt_s)

        @pl.when(s == NS - 1)
        def _():
            pltpu.sync_copy(zr_hbm.at[pl.ds(0, R_TAIL)],
                            acc_s.at[pl.ds(NS * R_MAIN, R_TAIL)])

        pltpu.sync_copy(ones_hbm, ones_v)
        e0 = pl.multiple_of(tid * E_PER_TILE, 8)
        pltpu.sync_copy(src_hbm.at[pl.ds(e0, E_PER_TILE)], src_a)
        pltpu.sync_copy(dst_hbm.at[pl.ds(e0, E_PER_TILE)], dst_a)
        plsc.subcore_barrier()

        def src_at(j):
            return src_a.at[pl.ds(pl.multiple_of(j * CHUNK, 8), CHUNK)]

        def dst_at(j):
            return dst_a.at[pl.ds(pl.multiple_of(j * CHUNK, 8), CHUNK)]

        def gather_wait(j, buf, sem):
            pltpu.make_async_copy(feat_hbm.at[src_at(j)], buf, sem).wait()

        def scatter_start(j, buf, semv, semc):
            pltpu.async_copy(buf, acc_s.at[dst_at(j)], semv, add=True)
            pltpu.async_copy(ones_v, cnt_s.at[dst_at(j)], semc, add=True)

        def scatter_wait(j, buf, semv, semc):
            pltpu.make_async_copy(buf, acc_s.at[dst_at(j)], semv).wait()
            pltpu.make_async_copy(ones_v, cnt_s.at[dst_at(j)], semc).wait()

        pltpu.async_copy(feat_hbm.at[src_at(0)], msgs0, sem0)
        gather_wait(0, msgs0, sem0)
        scatter_start(0, msgs0, sems0, semc0)
        pltpu.async_copy(feat_hbm.at[src_at(1)], msgs1, sem1)

        def body(j2, carry):
            j = j2 * 2 + 1
            gather_wait(j, msgs1, sem1)
            scatter_start(j, msgs1, sems1, semc1)
            scatter_wait(j - 1, msgs0, sems0, semc0)
            pltpu.async_copy(feat_hbm.at[src_at(j + 1)], msgs0, sem0)
            gather_wait(j + 1, msgs0, sem0)
            scatter_start(j + 1, msgs0, sems0, semc0)
            scatter_wait(j, msgs1, sems1, semc1)

            @pl.when(j2 < N_CHUNKS // 2 - 1)
            def _():
                pltpu.async_copy(feat_hbm.at[src_at(j + 2)], msgs1, sem1)

            return carry

        lax.fori_loop(0, N_CHUNKS // 2, body, 0)
        et0 = pl.multiple_of(N_CHUNKS * CHUNK, 8)
        pltpu.async_copy(feat_hbm.at[src_a.at[pl.ds(et0, E_TAIL)]],
                         msgs1.at[pl.ds(0, E_TAIL)], sem1)
        scatter_wait(N_CHUNKS - 1, msgs0, sems0, semc0)
        pltpu.make_async_copy(feat_hbm.at[src_a.at[pl.ds(et0, E_TAIL)]],
                              msgs1.at[pl.ds(0, E_TAIL)], sem1).wait()
        pltpu.sync_copy(msgs1.at[pl.ds(0, E_TAIL)],
                        acc_s.at[dst_a.at[pl.ds(et0, E_TAIL)]], add=True)
        pltpu.sync_copy(ones_v.at[pl.ds(0, E_TAIL)],
                        cnt_s.at[dst_a.at[pl.ds(et0, E_TAIL)]], add=True)

        plsc.subcore_barrier()

        pltpu.sync_copy(acc_s.at[pl.ds(r0, R_MAIN)],
                        psum_hbm.at[c, pl.ds(r0, R_MAIN)])

        @pl.when(s == 0)
        def _():
            pltpu.sync_copy(cnt_s, pcnt_hbm.at[pl.multiple_of(8 * c, 8)])

        @pl.when(s == NS - 1)
        def _():
            pltpu.sync_copy(acc_s.at[pl.ds(NS * R_MAIN, R_TAIL)],
                            psum_hbm.at[c, pl.ds(NS * R_MAIN, R_TAIL)])

    return k(feature, src, dst, zrows, zcnt, ones_v_h)


def _tc_body(p_ref, c_ref, w_ref, b_ref, o_ref):
    p = p_ref[0] + p_ref[1]
    cnt = jnp.maximum(c_ref[0] + c_ref[8], 1.0).reshape(N_NODES, 1)
    h = p / cnt
    o_ref[...] = (
        jnp.dot(h, w_ref[...], preferred_element_type=jnp.float32) + b_ref[...]
    )


def _tc_apply(psum, pcnt, Wt, b2):
    return pl.pallas_call(
        _tc_body,
        out_shape=jax.ShapeDtypeStruct((N_NODES, DIM), jnp.float32),
    )(psum, pcnt, Wt, b2)


def kernel(feature, edge_index, W, b):
    src = edge_index[0].astype(jnp.int32)
    dst = edge_index[1].astype(jnp.int32)
    zrows = jnp.zeros((R_MAIN, DIM), jnp.float32)
    zcnt = jnp.zeros((N_NODES,), jnp.float32)
    ones_v_h = jnp.ones((CHUNK,), jnp.float32)
    psum, pcnt = _sc_accumulate(feature, src, dst, zrows, zcnt, ones_v_h)
    out = _tc_apply(psum, pcnt, W.T, b.reshape(1, DIM))
    return out
